# Initial kernel scaffold; baseline (speedup 1.0000x reference)
import jax
import jax.numpy as jnp
from jax.experimental import pallas as pl


def kernel(vision, proprio, W_vision, b_vision, W_proprio, b_proprio, ids_vision, ids_proprio):
    raise NotImplementedError("write your pallas kernel here")


# speedup vs baseline: 1.5403x; 1.5403x over previous
import functools

import jax
import jax.numpy as jnp
from jax.experimental import pallas as pl
from jax.experimental.pallas import tpu as pltpu

_N_NEURON = 8192
_KV = 2048
_KP = 1024
_K = _KV + _KP
_MBLK = 512
_KB = 512


def _body(ids_ref, vis_ref, prp_ref, wv_ref, bv_ref, wp_ref, bp_ref, out_ref,
          src_ref):
    sv = jnp.dot(vis_ref[...], wv_ref[...], preferred_element_type=jnp.float32)
    sv = jnp.maximum(sv + bv_ref[...], 0.0)
    sp = jnp.dot(prp_ref[...], wp_ref[...], preferred_element_type=jnp.float32)
    sp = jnp.maximum(sp + bp_ref[...], 0.0)
    src_ref[:, :_KV] = sv.astype(jnp.bfloat16)
    src_ref[:, _KV:] = sp.astype(jnp.bfloat16)

    out_ref[...] = jnp.zeros_like(out_ref)

    def kb_step(i, _):
        idb = ids_ref[pl.ds(i * _KB, _KB), :]
        cols = jax.lax.broadcasted_iota(jnp.int32, (_KB, _N_NEURON), 1)
        onehot = (idb == cols).astype(jnp.bfloat16)
        sb = src_ref[:, pl.ds(i * _KB, _KB)]
        out_ref[...] += jnp.dot(sb, onehot, preferred_element_type=jnp.float32)
        return 0

    jax.lax.fori_loop(0, _K // _KB, kb_step, 0)


def kernel(vision, proprio, W_vision, b_vision, W_proprio, b_proprio,
           ids_vision, ids_proprio):
    B = vision.shape[0]
    ids = jnp.concatenate([ids_vision, ids_proprio]).reshape(_K, 1)
    grid = (B // _MBLK,)
    return pl.pallas_call(
        _body,
        grid=grid,
        in_specs=[
            pl.BlockSpec((_K, 1), lambda i: (0, 0)),
            pl.BlockSpec((_MBLK, 1024), lambda i: (i, 0)),
            pl.BlockSpec((_MBLK, 512), lambda i: (i, 0)),
            pl.BlockSpec((1024, _KV), lambda i: (0, 0)),
            pl.BlockSpec((1, _KV), lambda i: (0, 0)),
            pl.BlockSpec((512, _KP), lambda i: (0, 0)),
            pl.BlockSpec((1, _KP), lambda i: (0, 0)),
        ],
        out_specs=pl.BlockSpec((_MBLK, _N_NEURON), lambda i: (i, 0)),
        out_shape=jax.ShapeDtypeStruct((B, _N_NEURON), jnp.float32),
        scratch_shapes=[pltpu.VMEM((_MBLK, _K), jnp.bfloat16)],
    )(ids, vision, proprio, W_vision, b_vision.reshape(1, _KV),
      W_proprio, b_proprio.reshape(1, _KP))
